# baseline (speedup 1.0000x reference)
import jax
import jax.numpy as jnp
from jax import lax
from jax.experimental import pallas as pl
from jax.experimental.pallas import tpu as pltpu
from jax.experimental.pallas import tpu_sc as plsc

_V = 1000000
_D = 64
_B = 16384 * 50
_NPAIR = 3907
_PROWS = _NPAIR * 128
_TAIL = 999936

_info = plsc.get_sparse_core_info()
_NC, _NS = _info.num_cores, _info.num_subcores
_NW = _NC * _NS
_B_PER_W = _B // _NW
_CHUNK = 640
_GATHER = 128
_N_CHUNKS = _B_PER_W // _CHUNK


def _pack_body(in_ref, out_ref):
    out_ref[...] = jnp.concatenate(
        [in_ref[:, 0:128].T, in_ref[:, 128:256].T], axis=1)


@jax.jit
def _pack(wt):
    return pl.pallas_call(
        _pack_body,
        grid=(_NPAIR,),
        in_specs=[pl.BlockSpec((_D, 256), lambda i: (0, i))],
        out_specs=pl.BlockSpec((128, 128), lambda i: (i, 0)),
        out_shape=jax.ShapeDtypeStruct((_PROWS, 128), jnp.float32),
    )(wt)


def _gather_body(table_hbm, idx_hbm, out_hbm, idx_v, idxj_v, rows_v, sem):
    wid = lax.axis_index("s") * _NC + lax.axis_index("c")
    base = wid * _B_PER_W

    def chunk(i, carry):
        off = base + i * _CHUNK
        pltpu.sync_copy(idx_hbm.at[pl.ds(off, _CHUNK)], idx_v)
        for v in range(_CHUNK // 16):
            r = idx_v[pl.ds(v * 16, 16)]
            j = ((r >> 8) << 8) + ((r & 127) << 1) + ((r >> 7) & 1)
            j = jnp.where(r >= _TAIL, r * 2 - 999936, j)
            idxj_v[pl.ds(v * 16, 16)] = j
        copies = []
        for g in range(_CHUNK // _GATHER):
            copies.append(
                pltpu.async_copy(
                    table_hbm.at[idxj_v.at[pl.ds(g * _GATHER, _GATHER)]],
                    rows_v.at[pl.ds(g * _GATHER, _GATHER)], sem))
        for c in copies:
            c.wait()
        pltpu.sync_copy(rows_v, out_hbm.at[pl.ds(off, _CHUNK)])
        return carry

    lax.fori_loop(0, _N_CHUNKS, chunk, 0)


@jax.jit
def _lookup(w64, idx):
    mesh = plsc.VectorSubcoreMesh(core_axis_name="c", subcore_axis_name="s")
    f = pl.kernel(
        _gather_body,
        mesh=mesh,
        out_type=jax.ShapeDtypeStruct((_B, _D), jnp.float32),
        scratch_types=[
            pltpu.VMEM((_CHUNK,), jnp.int32),
            pltpu.VMEM((_CHUNK,), jnp.int32),
            pltpu.VMEM((_CHUNK, _D), jnp.float32),
            pltpu.SemaphoreType.DMA,
        ],
        compiler_params=pltpu.CompilerParams(use_tc_tiling_on_sc=False),
    )
    return f(w64, idx)


def kernel(input_, weight):
    wt = weight.T
    packed = _pack(wt)
    w64 = packed.reshape(-1, _D)
    idx = input_.reshape(-1).astype(jnp.int32)
    out = _lookup(w64, idx)
    return out.reshape(input_.shape + (weight.shape[-1],))


# speedup vs baseline: 2.5960x; 2.5960x over previous
import jax
import jax.numpy as jnp
from jax import lax
from jax.experimental import pallas as pl
from jax.experimental.pallas import tpu as pltpu
from jax.experimental.pallas import tpu_sc as plsc

_V = 1000000
_D = 64
_B = 16384 * 50
_PPB = 10
_CPB = 256 * _PPB
_NBLK = 391
_PROWS = _NBLK * _PPB * 128
_TAIL = 999936

_info = plsc.get_sparse_core_info()
_NC, _NS = _info.num_cores, _info.num_subcores
_NW = _NC * _NS
_B_PER_W = _B // _NW
_CHUNK = 640
_GATHER = 128
_N_CHUNKS = _B_PER_W // _CHUNK


def _pack_body(in_ref, out_ref):
    eye = jnp.float32(1) * (
        lax.broadcasted_iota(jnp.int32, (_D, _D), 0)
        == lax.broadcasted_iota(jnp.int32, (_D, _D), 1))
    at = lax.dot_general(in_ref[...], eye, (((0,), (0,)), ((), ())),
                         preferred_element_type=jnp.float32)
    for j in range(_PPB):
        out_ref[pl.ds(j * 128, 128), :] = jnp.concatenate(
            [at[j * 256:j * 256 + 128, :], at[j * 256 + 128:j * 256 + 256, :]],
            axis=1)


@jax.jit
def _pack(wt):
    return pl.pallas_call(
        _pack_body,
        grid=(_NBLK,),
        in_specs=[pl.BlockSpec((_D, _CPB), lambda i: (0, i))],
        out_specs=pl.BlockSpec((_PPB * 128, 128), lambda i: (i, 0)),
        out_shape=jax.ShapeDtypeStruct((_PROWS, 128), jnp.float32),
    )(wt)


def _gather_body(table_hbm, idx_hbm, out_hbm, idx_v, idxj_v, rows_v, sem):
    wid = lax.axis_index("s") * _NC + lax.axis_index("c")
    base = wid * _B_PER_W

    def chunk(i, carry):
        off = base + i * _CHUNK
        pltpu.sync_copy(idx_hbm.at[pl.ds(off, _CHUNK)], idx_v)
        for v in range(_CHUNK // 16):
            r = idx_v[pl.ds(v * 16, 16)]
            j = ((r >> 8) << 8) + ((r & 127) << 1) + ((r >> 7) & 1)
            j = jnp.where(r >= _TAIL, r * 2 - 999936, j)
            idxj_v[pl.ds(v * 16, 16)] = j
        copies = []
        for g in range(_CHUNK // _GATHER):
            copies.append(
                pltpu.async_copy(
                    table_hbm.at[idxj_v.at[pl.ds(g * _GATHER, _GATHER)]],
                    rows_v.at[pl.ds(g * _GATHER, _GATHER)], sem))
        for c in copies:
            c.wait()
        pltpu.sync_copy(rows_v, out_hbm.at[pl.ds(off, _CHUNK)])
        return carry

    lax.fori_loop(0, _N_CHUNKS, chunk, 0)


@jax.jit
def _lookup(w64, idx):
    mesh = plsc.VectorSubcoreMesh(core_axis_name="c", subcore_axis_name="s")
    f = pl.kernel(
        _gather_body,
        mesh=mesh,
        out_type=jax.ShapeDtypeStruct((_B, _D), jnp.float32),
        scratch_types=[
            pltpu.VMEM((_CHUNK,), jnp.int32),
            pltpu.VMEM((_CHUNK,), jnp.int32),
            pltpu.VMEM((_CHUNK, _D), jnp.float32),
            pltpu.SemaphoreType.DMA,
        ],
        compiler_params=pltpu.CompilerParams(use_tc_tiling_on_sc=False),
    )
    return f(w64, idx)


def kernel(input_, weight):
    wt = weight.T
    packed = _pack(wt)
    w64 = packed.reshape(-1, _D)
    idx = input_.reshape(-1).astype(jnp.int32)
    out = _lookup(w64, idx)
    return out.reshape(input_.shape + (weight.shape[-1],))
